# Initial kernel scaffold; baseline (speedup 1.0000x reference)
import jax
import jax.numpy as jnp
from jax.experimental import pallas as pl


def kernel(h, idx):
    raise NotImplementedError("write your pallas kernel here")


# speedup vs baseline: 2.8367x; 2.8367x over previous
import functools

import jax
import jax.numpy as jnp
from jax import lax
from jax.experimental import pallas as pl
from jax.experimental.pallas import tpu as pltpu
from jax.experimental.pallas import tpu_sc as plsc

R = 2.0
T = 1.0

NC = 2
NS = 16
NW = NC * NS
L = 16

N_NODES = 10000
D = 128
N_EDGES = 320000
E_W = N_EDGES // NW
CHUNK = 80
NCH = E_W // CHUNK


def _renorm_tc(h):
    blk = 1000

    def body(h_ref, o_ref):
        x = h_ref[...]
        ss = jnp.sum(x * x, axis=1, keepdims=True)
        norm = jnp.sqrt(ss)
        scale = jnp.where(norm > 1.0, 1.0 / jnp.maximum(norm, 1e-12), 1.0)
        o_ref[...] = x * scale

    return pl.pallas_call(
        body,
        out_shape=jax.ShapeDtypeStruct((N_NODES, D), jnp.float32),
        grid=(N_NODES // blk,),
        in_specs=[pl.BlockSpec((blk, D), lambda i: (i, 0))],
        out_specs=pl.BlockSpec((blk, D), lambda i: (i, 0)),
    )(h)


def _decode_sc(hr, idx0, idx1):
    mesh = plsc.VectorSubcoreMesh(core_axis_name="c", subcore_axis_name="s")

    @functools.partial(
        pl.kernel,
        out_type=jax.ShapeDtypeStruct((N_EDGES,), jnp.float32),
        mesh=mesh,
        scratch_types=[
            pltpu.VMEM((CHUNK,), jnp.int32),
            pltpu.VMEM((CHUNK,), jnp.int32),
            pltpu.VMEM((CHUNK, D), jnp.float32),
            pltpu.VMEM((CHUNK, D), jnp.float32),
            pltpu.VMEM((CHUNK,), jnp.float32),
            pltpu.SemaphoreType.DMA,
        ],
    )
    def decode(hr_hbm, idx0_hbm, idx1_hbm, out_hbm,
               idx0_v, idx1_v, rows0_v, rows1_v, out_v, sem):
        wid = lax.axis_index("s") * NC + lax.axis_index("c")
        base = wid * E_W

        def chunk_body(ci, carry):
            cbase = base + ci * CHUNK
            pltpu.sync_copy(idx0_hbm.at[pl.ds(cbase, CHUNK)], idx0_v)
            pltpu.sync_copy(idx1_hbm.at[pl.ds(cbase, CHUNK)], idx1_v)
            cp0 = pltpu.async_copy(hr_hbm.at[idx0_v], rows0_v, sem)
            cp1 = pltpu.async_copy(hr_hbm.at[idx1_v], rows1_v, sem)
            cp0.wait()
            cp1.wait()

            lane = lax.iota(jnp.int32, L)
            perms = [(lane ^ (1 << k))[:, None] for k in range(4)]
            dnums = lax.GatherDimensionNumbers(
                offset_dims=(), collapsed_slice_dims=(0,),
                start_index_map=(0,))

            def lane_sum(v):
                for p in perms:
                    v = v + lax.gather(
                        v, p, dnums, slice_sizes=(1,),
                        mode=lax.GatherScatterMode.PROMISE_IN_BOUNDS)
                return v

            def group_body(g, c2):
                res = jnp.zeros((L,), jnp.float32)
                for k in range(L):
                    e = g * L + k
                    acc = jnp.zeros((L,), jnp.float32)
                    for d in range(D // L):
                        a = rows0_v[e, pl.ds(d * L, L)]
                        b = rows1_v[e, pl.ds(d * L, L)]
                        df = a - b
                        acc = acc + df * df
                    res = jnp.where(lane == k, lane_sum(acc), res)
                out_v[pl.ds(g * L, L)] = 1.0 / (jnp.exp((res - R) / T) + 1.0)
                return c2

            lax.fori_loop(0, CHUNK // L, group_body, 0, unroll=False)

            pltpu.sync_copy(out_v, out_hbm.at[pl.ds(cbase, CHUNK)])
            return carry

        lax.fori_loop(0, NCH, chunk_body, 0, unroll=False)

    return decode(hr, idx0, idx1)


def kernel(h, idx):
    idx = idx.astype(jnp.int32)
    idx0 = idx[:, 0]
    idx1 = idx[:, 1]
    hr = _renorm_tc(h)
    return _decode_sc(hr, idx0, idx1)


# speedup vs baseline: 5.3111x; 1.8723x over previous
import functools

import jax
import jax.numpy as jnp
from jax import lax
from jax.experimental import pallas as pl
from jax.experimental.pallas import tpu as pltpu
from jax.experimental.pallas import tpu_sc as plsc

R = 2.0
T = 1.0

NC = 2
NS = 16
NW = NC * NS
L = 16

N_NODES = 10000
D = 128
N_EDGES = 320000
E_W = N_EDGES // NW
CHUNK = 80
NCH = E_W // CHUNK


def _renorm_tc(h):
    blk = 1000

    def body(h_ref, o_ref):
        x = h_ref[...]
        ss = jnp.sum(x * x, axis=1, keepdims=True)
        norm = jnp.sqrt(ss)
        scale = jnp.where(norm > 1.0, 1.0 / jnp.maximum(norm, 1e-12), 1.0)
        o_ref[...] = x * scale

    return pl.pallas_call(
        body,
        out_shape=jax.ShapeDtypeStruct((N_NODES, D), jnp.float32),
        grid=(N_NODES // blk,),
        in_specs=[pl.BlockSpec((blk, D), lambda i: (i, 0))],
        out_specs=pl.BlockSpec((blk, D), lambda i: (i, 0)),
    )(h)


def _decode_sc(hr, idx0, idx1):
    mesh = plsc.VectorSubcoreMesh(core_axis_name="c", subcore_axis_name="s")

    @functools.partial(
        pl.kernel,
        out_type=jax.ShapeDtypeStruct((N_EDGES,), jnp.float32),
        mesh=mesh,
        scratch_types=[
            pltpu.VMEM((E_W,), jnp.int32),
            pltpu.VMEM((E_W,), jnp.int32),
            pltpu.VMEM((CHUNK, D), jnp.float32),
            pltpu.VMEM((CHUNK, D), jnp.float32),
            pltpu.VMEM((CHUNK, D), jnp.float32),
            pltpu.VMEM((CHUNK, D), jnp.float32),
            pltpu.VMEM((E_W,), jnp.float32),
            pltpu.SemaphoreType.DMA,
            pltpu.SemaphoreType.DMA,
        ],
    )
    def decode(hr_hbm, idx0_hbm, idx1_hbm, out_hbm,
               idx0_all, idx1_all, rows0_a, rows1_a, rows0_b, rows1_b,
               out_all, sem_a, sem_b):
        wid = lax.axis_index("s") * NC + lax.axis_index("c")
        base = wid * E_W

        pltpu.sync_copy(idx0_hbm.at[pl.ds(base, E_W)], idx0_all)
        pltpu.sync_copy(idx1_hbm.at[pl.ds(base, E_W)], idx1_all)

        def start(ci, r0, r1, sem):
            off = ci * CHUNK
            pltpu.async_copy(hr_hbm.at[idx0_all.at[pl.ds(off, CHUNK)]], r0, sem)
            pltpu.async_copy(hr_hbm.at[idx1_all.at[pl.ds(off, CHUNK)]], r1, sem)

        def wait(r0, r1, sem):
            pltpu.make_async_copy(hr_hbm.at[pl.ds(0, CHUNK)], r0, sem).wait()
            pltpu.make_async_copy(hr_hbm.at[pl.ds(0, CHUNK)], r1, sem).wait()

        lane = lax.iota(jnp.int32, L)
        perms = [(lane ^ (1 << k))[:, None] for k in range(4)]
        dnums = lax.GatherDimensionNumbers(
            offset_dims=(), collapsed_slice_dims=(0,),
            start_index_map=(0,))

        def lane_sum(v):
            for p in perms:
                v = v + lax.gather(
                    v, p, dnums, slice_sizes=(1,),
                    mode=lax.GatherScatterMode.PROMISE_IN_BOUNDS)
            return v

        def compute(ci, r0, r1):
            obase = ci * CHUNK

            def group_body(g, c2):
                res = jnp.zeros((L,), jnp.float32)
                for k in range(L):
                    e = g * L + k
                    acc = jnp.zeros((L,), jnp.float32)
                    for d in range(D // L):
                        a = r0[e, pl.ds(d * L, L)]
                        b = r1[e, pl.ds(d * L, L)]
                        df = a - b
                        acc = acc + df * df
                    res = jnp.where(lane == k, lane_sum(acc), res)
                out_all[pl.ds(obase + g * L, L)] = (
                    1.0 / (jnp.exp((res - R) / T) + 1.0))
                return c2

            lax.fori_loop(0, CHUNK // L, group_body, 0, unroll=False)

        start(0, rows0_a, rows1_a, sem_a)

        def pair_body(gg, carry):
            c0 = 2 * gg
            c1 = c0 + 1

            @pl.when(c1 < NCH)
            def _():
                start(c1, rows0_b, rows1_b, sem_b)

            wait(rows0_a, rows1_a, sem_a)
            compute(c0, rows0_a, rows1_a)

            @pl.when(c0 + 2 < NCH)
            def _():
                start(c0 + 2, rows0_a, rows1_a, sem_a)

            @pl.when(c1 < NCH)
            def _():
                wait(rows0_b, rows1_b, sem_b)
                compute(c1, rows0_b, rows1_b)

            return carry

        lax.fori_loop(0, (NCH + 1) // 2, pair_body, 0, unroll=False)

        pltpu.sync_copy(out_all, out_hbm.at[pl.ds(base, E_W)])

    return decode(hr, idx0, idx1)


def kernel(h, idx):
    idx = idx.astype(jnp.int32)
    idx0 = idx[:, 0]
    idx1 = idx[:, 1]
    hr = _renorm_tc(h)
    return _decode_sc(hr, idx0, idx1)
